# baseline (speedup 1.0000x reference)
import jax
import jax.numpy as jnp
from jax.experimental import pallas as pl
from jax.experimental.pallas import tpu as pltpu

_BS = 512


def _pairswap(v, even):
    return jnp.where(even, jnp.roll(v, -1, axis=-1), jnp.roll(v, 1, axis=-1))


def _add_pe_kernel(x_ref, base_ref, ph_ref, o_ref, u_ref, v_ref, pt_ref):
    d = x_ref.shape[-1]
    s = pl.program_id(0)
    b = pl.program_id(1)
    even = (jax.lax.broadcasted_iota(jnp.int32, (1, d), 1) % 2) == 0

    @pl.when((s == 0) & (b == 0))
    def _init():
        a = base_ref[0]
        bsw = _pairswap(a, even)
        u_ref[...] = jnp.where(even, bsw, a)
        v_ref[...] = jnp.where(even, a, -bsw)

    @pl.when(b == 0)
    def _new_tile():
        p = ph_ref[0, 0:1]
        q = _pairswap(p, even)
        pt_ref[...] = p * u_ref[...] + q * v_ref[...]

    o_ref[0] = x_ref[0] + pt_ref[...]


def kernel(x, pe):
    b, seq_len, d = x.shape
    n_tiles = seq_len // _BS
    return pl.pallas_call(
        _add_pe_kernel,
        grid=(n_tiles, b),
        in_specs=[
            pl.BlockSpec((1, _BS, d), lambda s, i: (i, s, 0)),
            pl.BlockSpec((1, _BS, d), lambda s, i: (0, 0, 0)),
            pl.BlockSpec((1, 8, d), lambda s, i: (0, s * (_BS // 8), 0)),
        ],
        out_specs=pl.BlockSpec((1, _BS, d), lambda s, i: (i, s, 0)),
        out_shape=jax.ShapeDtypeStruct((b, seq_len, d), x.dtype),
        scratch_shapes=[
            pltpu.VMEM((_BS, d), jnp.float32),
            pltpu.VMEM((_BS, d), jnp.float32),
            pltpu.VMEM((_BS, d), jnp.float32),
        ],
    )(x, pe, pe)


# speedup vs baseline: 1.1815x; 1.1815x over previous
import jax
import jax.numpy as jnp
from jax.experimental import pallas as pl
from jax.experimental.pallas import tpu as pltpu

_BS = 512


def _pairswap(v, even):
    return jnp.where(even, jnp.roll(v, -1, axis=-1), jnp.roll(v, 1, axis=-1))


def _add_pe_kernel(x_ref, base_ref, ph_ref, o_ref, u_ref, v_ref):
    d = x_ref.shape[-1]
    even = (jax.lax.broadcasted_iota(jnp.int32, (1, d), 1) % 2) == 0

    @pl.when(pl.program_id(0) == 0)
    def _init():
        a = base_ref[0]
        bsw = _pairswap(a, even)
        u_ref[...] = jnp.where(even, bsw, a)
        v_ref[...] = jnp.where(even, a, -bsw)

    p = ph_ref[0, 0:1]
    q = _pairswap(p, even)
    o_ref[...] = x_ref[...] + (p * u_ref[...] + q * v_ref[...])[None]


def kernel(x, pe):
    b, seq_len, d = x.shape
    n_tiles = seq_len // _BS
    return pl.pallas_call(
        _add_pe_kernel,
        grid=(n_tiles,),
        in_specs=[
            pl.BlockSpec((b, _BS, d), lambda s: (0, s, 0)),
            pl.BlockSpec((1, _BS, d), lambda s: (0, 0, 0)),
            pl.BlockSpec((1, 8, d), lambda s: (0, s * (_BS // 8), 0)),
        ],
        out_specs=pl.BlockSpec((b, _BS, d), lambda s: (0, s, 0)),
        out_shape=jax.ShapeDtypeStruct((b, seq_len, d), x.dtype),
        scratch_shapes=[
            pltpu.VMEM((_BS, d), jnp.float32),
            pltpu.VMEM((_BS, d), jnp.float32),
        ],
    )(x, pe, pe)
